# baseline (speedup 1.0000x reference)
import jax
import jax.numpy as jnp
from jax import lax
from jax.experimental import pallas as pl
from jax.experimental.pallas import tpu as pltpu
from jax.experimental.pallas import tpu_sc as plsc

N = 10000
C = 256
H = 8
D = 32
KV = 27
P = 160000

NCORE = 2
NSUB = 16
LANES = 16
HC = C // NCORE
HPC = H // NCORE
PPT = P // NSUB
PPTP = 10240
CB = 32
NCHUNK = PPTP // CB
SB = 8
NSB = NCHUNK // SB
NGRP = CB // LANES
NP = 10112
RPT = NP // NSUB
RBLK = 1000
NRB = N // RBLK
EPS = 1e-12


def _proj_body(x_ref, wq_ref, bq_ref, wk_ref, bk_ref, wv_ref, bv_ref,
               pos_ref, m_ref, nq_ref, kv_ref, npos_ref):
    i = pl.program_id(1)
    xb = x_ref[...]
    m = m_ref[...]

    def norm(a):
        s = jnp.dot(a * a, m, preferred_element_type=jnp.float32)
        return a / jnp.maximum(jnp.sqrt(s), EPS)

    bq = bq_ref[...].reshape(1, HC)
    bk = bk_ref[...].reshape(1, HC)
    bv = bv_ref[...].reshape(1, HC)
    q = jnp.dot(xb, wq_ref[...], preferred_element_type=jnp.float32) + bq
    k = jnp.dot(xb, wk_ref[...], preferred_element_type=jnp.float32) + bk
    v = jnp.dot(xb, wv_ref[...], preferred_element_type=jnp.float32) + bv
    nq_ref[...] = norm(q).reshape(1, RBLK, HC)
    kv_ref[0, :, :HC] = norm(k)
    kv_ref[0, :, HC:] = v

    @pl.when(i == 0)
    def _():
        p = pos_ref[...]
        npos_ref[...] = norm(p).reshape(1, KV, HC)


def _proj(x, wq, bq2, wk, bk2, wv, bv2, pos, m):
    f32 = jnp.float32
    return pl.pallas_call(
        _proj_body,
        grid=(NCORE, NRB),
        in_specs=[
            pl.BlockSpec((RBLK, C), lambda hh, i: (i, 0)),
            pl.BlockSpec((C, HC), lambda hh, i: (0, hh)),
            pl.BlockSpec((1, 1, HC), lambda hh, i: (hh, 0, 0)),
            pl.BlockSpec((C, HC), lambda hh, i: (0, hh)),
            pl.BlockSpec((1, 1, HC), lambda hh, i: (hh, 0, 0)),
            pl.BlockSpec((C, HC), lambda hh, i: (0, hh)),
            pl.BlockSpec((1, 1, HC), lambda hh, i: (hh, 0, 0)),
            pl.BlockSpec((KV, HC), lambda hh, i: (0, hh)),
            pl.BlockSpec((HC, HC), lambda hh, i: (0, 0)),
        ],
        out_specs=[
            pl.BlockSpec((1, RBLK, HC), lambda hh, i: (hh, i, 0)),
            pl.BlockSpec((1, RBLK, 2 * HC), lambda hh, i: (hh, i, 0)),
            pl.BlockSpec((1, KV, HC), lambda hh, i: (hh, 0, 0)),
        ],
        out_shape=[
            jax.ShapeDtypeStruct((NCORE, N, HC), f32),
            jax.ShapeDtypeStruct((NCORE, N, 2 * HC), f32),
            jax.ShapeDtypeStruct((NCORE, KV, HC), f32),
        ],
    )(x, wq, bq2, wk, bk2, wv, bv2, pos, m)


def _sc_body(nq_hbm, kv_hbm, npos_hbm, idx_hbm, zz_hbm, o_hbm,
             qrA, qrB, kvA, kvB, ct, nposv, idxbA, idxbB,
             acc, gsA, gsB, ss, isem):
    hh = lax.axis_index("c")
    s = lax.axis_index("s")
    nq_t = nq_hbm.at[hh]
    kv_t = kv_hbm.at[hh]
    pltpu.sync_copy(npos_hbm.at[hh], nposv)
    pltpu.sync_copy(zz_hbm.at[pl.ds(s * RPT, RPT)],
                    acc.at[pl.ds(s * RPT, RPT)])
    plsc.subcore_barrier()

    iota = lax.iota(jnp.int32, LANES)

    def issue_gather(idxb, j, qr, kv, gs):
        pltpu.async_copy(nq_t.at[idxb.at[1, j]], qr, gs)
        pltpu.async_copy(kv_t.at[idxb.at[0, j]], kv, gs)

    def wait_gather(idxb, j, qr, kv, gs):
        pltpu.make_async_copy(nq_t.at[idxb.at[1, j]], qr, gs).wait()
        pltpu.make_async_copy(kv_t.at[idxb.at[0, j]], kv, gs).wait()

    def wait_scatter(ct, ss):
        pltpu.make_async_copy(zz_hbm.at[pl.ds(0, CB)], ct, ss).wait()

    def compute(idxb, j, qr, kv, ct):
        def group(g, c2):
            pvec = iota + g * LANES
            kvec = idxb[3, j, pl.ds(pl.multiple_of(g * LANES, LANES), LANES)]

            def head(h, c3):
                base = h * D
                accv = jnp.zeros((LANES,), jnp.float32)
                for cc in range(D):
                    col = ((iota + cc) & (D - 1)) + base
                    vq = plsc.load_gather(qr, [pvec, col])
                    vk = plsc.load_gather(kv, [pvec, col])
                    vp = plsc.load_gather(nposv, [kvec, col])
                    accv = accv + vq * (vk + vp)
                for cc in range(D):
                    col2 = ((iota + cc) & (D - 1)) + base
                    colv = col2 + HC
                    vv = plsc.load_gather(kv, [pvec, colv])
                    plsc.store_scatter(ct, [pvec, col2], accv * vv)
                return c3

            lax.fori_loop(0, HPC, head, 0)
            return c2

        lax.fori_loop(0, NGRP, group, 0)

    bufs = ((qrA, kvA, gsA), (qrB, kvB, gsB))
    ibufs = (idxbA, idxbB)

    pltpu.sync_copy(idx_hbm.at[s, 0], idxbA)
    issue_gather(idxbA, 0, qrA, kvA, gsA)

    def super_pair(sb2, carry):
        for p in range(2):
            sb = sb2 * 2 + p
            idxb = ibufs[p]
            oidxb = ibufs[1 - p]

            @pl.when(sb < NSB - 1)
            def _():
                pltpu.async_copy(idx_hbm.at[s, sb + 1], oidxb, isem)

            def inner(j2, c):
                for b in range(2):
                    qr, kv, gs = bufs[b]
                    oqr, okv, ogs = bufs[1 - b]
                    j = j2 * 2 + b
                    wait_gather(idxb, j, qr, kv, gs)
                    if b == 0:
                        issue_gather(idxb, j + 1, oqr, okv, ogs)
                    else:
                        @pl.when(j2 < SB // 2 - 1)
                        def _():
                            issue_gather(idxb, j + 1, oqr, okv, ogs)

                    if b == 1:
                        wait_scatter(ct, ss)
                    else:
                        @pl.when((j2 >= 1) | (sb >= 1))
                        def _():
                            wait_scatter(ct, ss)

                    compute(idxb, j, qr, kv, ct)
                    pltpu.async_copy(ct, acc.at[idxb.at[2, j]], ss, add=True)
                return c

            lax.fori_loop(0, SB // 2, inner, 0)

            @pl.when(sb < NSB - 1)
            def _():
                pltpu.make_async_copy(idx_hbm.at[s, sb + 1], oidxb,
                                      isem).wait()
                issue_gather(oidxb, 0, qrA, kvA, gsA)
        return carry

    lax.fori_loop(0, NSB // 2, super_pair, 0)
    wait_scatter(ct, ss)
    plsc.subcore_barrier()
    pltpu.sync_copy(acc.at[pl.ds(s * RPT, RPT)],
                    o_hbm.at[hh, pl.ds(s * RPT, RPT)])


def _attn(nq3, kv3, npos3, idx5, zz):
    f32 = jnp.float32
    fn = pl.kernel(
        _sc_body,
        out_type=jax.ShapeDtypeStruct((NCORE, NP, HC), f32),
        mesh=plsc.VectorSubcoreMesh(core_axis_name="c", subcore_axis_name="s"),
        compiler_params=pltpu.CompilerParams(needs_layout_passes=False),
        scratch_types=[
            pltpu.VMEM((CB, HC), f32),
            pltpu.VMEM((CB, HC), f32),
            pltpu.VMEM((CB, 2 * HC), f32),
            pltpu.VMEM((CB, 2 * HC), f32),
            pltpu.VMEM((CB, HC), f32),
            pltpu.VMEM((KV, HC), f32),
            pltpu.VMEM((4, SB, CB), jnp.int32),
            pltpu.VMEM((4, SB, CB), jnp.int32),
            pltpu.VMEM_SHARED((NP, HC), f32),
            pltpu.SemaphoreType.DMA,
            pltpu.SemaphoreType.DMA,
            pltpu.SemaphoreType.DMA,
            pltpu.SemaphoreType.DMA,
        ],
    )
    return fn(nq3, kv3, npos3, idx5, zz)


def _out_body(o0_ref, o1_ref, wo_ref, bo_ref, x_ref, y_ref):
    oc = jnp.concatenate([o0_ref[...].reshape(RBLK, HC),
                          o1_ref[...].reshape(RBLK, HC)], axis=1)
    y_ref[...] = (jnp.dot(oc, wo_ref[...], preferred_element_type=jnp.float32)
                  + bo_ref[...] + x_ref[...])


def _outproj(out2, wo, bo2, x):
    return pl.pallas_call(
        _out_body,
        grid=(NRB,),
        in_specs=[
            pl.BlockSpec((1, RBLK, HC), lambda i: (0, i, 0)),
            pl.BlockSpec((1, RBLK, HC), lambda i: (1, i, 0)),
            pl.BlockSpec((C, C), lambda i: (0, 0)),
            pl.BlockSpec((1, C), lambda i: (0, 0)),
            pl.BlockSpec((RBLK, C), lambda i: (i, 0)),
        ],
        out_specs=pl.BlockSpec((RBLK, C), lambda i: (i, 0)),
        out_shape=jax.ShapeDtypeStruct((N, C), jnp.float32),
    )(out2, out2, wo, bo2, x)


def kernel(x, kq_indices, kernel_idx, Wq, bq, Wk, bk, Wv, bv, Wo, bo, pos_enc):
    f32 = jnp.float32
    npad = PPTP - PPT
    in_t = kq_indices[0].reshape(NSUB, PPT)
    out_t = kq_indices[1].reshape(NSUB, PPT)
    kid_t = kernel_idx.reshape(NSUB, PPT)
    zpad = jnp.zeros((NSUB, npad), jnp.int32)
    in4 = jnp.concatenate([in_t, zpad], axis=1).reshape(NSUB, NSB, SB, CB)
    og4 = jnp.concatenate([out_t, zpad], axis=1).reshape(NSUB, NSB, SB, CB)
    os4 = jnp.concatenate([out_t, jnp.full((NSUB, npad), N, jnp.int32)],
                          axis=1).reshape(NSUB, NSB, SB, CB)
    kidx4 = jnp.concatenate([kid_t, zpad], axis=1).reshape(NSUB, NSB, SB, CB)
    idx5 = jnp.stack([in4, og4, os4, kidx4], axis=2)
    pos = pos_enc.reshape(KV, C)
    m = jnp.kron(jnp.eye(HPC, dtype=f32), jnp.ones((D, D), f32))
    nq2, kv2, npos2 = _proj(x, Wq, bq.reshape(NCORE, 1, HC), Wk,
                            bk.reshape(NCORE, 1, HC), Wv,
                            bv.reshape(NCORE, 1, HC), pos, m)
    zz = jnp.zeros((NP, HC), f32)
    out2 = _attn(nq2, kv2, npos2, idx5, zz)
    return _outproj(out2, Wo, bo.reshape(1, C), x)


# speedup vs baseline: 1.0034x; 1.0034x over previous
import jax
import jax.numpy as jnp
from jax import lax
from jax.experimental import pallas as pl
from jax.experimental.pallas import tpu as pltpu
from jax.experimental.pallas import tpu_sc as plsc

N = 10000
C = 256
H = 8
D = 32
KV = 27
P = 160000

NCORE = 2
NSUB = 16
LANES = 16
HC = C // NCORE
HPC = H // NCORE
PPT = P // NSUB
PPTP = 10240
CB = 32
NCHUNK = PPTP // CB
SB = 16
NSB = NCHUNK // SB
NGRP = CB // LANES
NP = 10112
RPT = NP // NSUB
RBLK = 1000
NRB = N // RBLK
EPS = 1e-12


def _proj_body(x_ref, wq_ref, bq_ref, wk_ref, bk_ref, wv_ref, bv_ref,
               pos_ref, m_ref, nq_ref, kv_ref, npos_ref):
    i = pl.program_id(1)
    xb = x_ref[...]
    m = m_ref[...]

    def norm(a):
        s = jnp.dot(a * a, m, preferred_element_type=jnp.float32)
        return a / jnp.maximum(jnp.sqrt(s), EPS)

    bq = bq_ref[...].reshape(1, HC)
    bk = bk_ref[...].reshape(1, HC)
    bv = bv_ref[...].reshape(1, HC)
    q = jnp.dot(xb, wq_ref[...], preferred_element_type=jnp.float32) + bq
    k = jnp.dot(xb, wk_ref[...], preferred_element_type=jnp.float32) + bk
    v = jnp.dot(xb, wv_ref[...], preferred_element_type=jnp.float32) + bv
    nq_ref[...] = norm(q).reshape(1, RBLK, HC)
    kv_ref[0, :, :HC] = norm(k)
    kv_ref[0, :, HC:] = v

    @pl.when(i == 0)
    def _():
        p = pos_ref[...]
        npos_ref[...] = norm(p).reshape(1, KV, HC)


def _proj(x, wq, bq2, wk, bk2, wv, bv2, pos, m):
    f32 = jnp.float32
    return pl.pallas_call(
        _proj_body,
        grid=(NCORE, NRB),
        in_specs=[
            pl.BlockSpec((RBLK, C), lambda hh, i: (i, 0)),
            pl.BlockSpec((C, HC), lambda hh, i: (0, hh)),
            pl.BlockSpec((1, 1, HC), lambda hh, i: (hh, 0, 0)),
            pl.BlockSpec((C, HC), lambda hh, i: (0, hh)),
            pl.BlockSpec((1, 1, HC), lambda hh, i: (hh, 0, 0)),
            pl.BlockSpec((C, HC), lambda hh, i: (0, hh)),
            pl.BlockSpec((1, 1, HC), lambda hh, i: (hh, 0, 0)),
            pl.BlockSpec((KV, HC), lambda hh, i: (0, hh)),
            pl.BlockSpec((HC, HC), lambda hh, i: (0, 0)),
        ],
        out_specs=[
            pl.BlockSpec((1, RBLK, HC), lambda hh, i: (hh, i, 0)),
            pl.BlockSpec((1, RBLK, 2 * HC), lambda hh, i: (hh, i, 0)),
            pl.BlockSpec((1, KV, HC), lambda hh, i: (hh, 0, 0)),
        ],
        out_shape=[
            jax.ShapeDtypeStruct((NCORE, N, HC), f32),
            jax.ShapeDtypeStruct((NCORE, N, 2 * HC), f32),
            jax.ShapeDtypeStruct((NCORE, KV, HC), f32),
        ],
    )(x, wq, bq2, wk, bk2, wv, bv2, pos, m)


def _sc_body(nq_hbm, kv_hbm, npos_hbm, idx_hbm, zz_hbm, o_hbm,
             qrA, qrB, kvA, kvB, ct, nposv, idxb,
             acc, gsA, gsB, ss):
    hh = lax.axis_index("c")
    s = lax.axis_index("s")
    nq_t = nq_hbm.at[hh]
    kv_t = kv_hbm.at[hh]
    pltpu.sync_copy(npos_hbm.at[hh], nposv)
    pltpu.sync_copy(zz_hbm.at[pl.ds(s * RPT, RPT)],
                    acc.at[pl.ds(s * RPT, RPT)])
    plsc.subcore_barrier()

    iota = lax.iota(jnp.int32, LANES)

    def issue_gather(j, qr, kv, gs):
        pltpu.async_copy(nq_t.at[idxb.at[1, j]], qr, gs)
        pltpu.async_copy(kv_t.at[idxb.at[0, j]], kv, gs)

    def wait_gather(j, qr, kv, gs):
        pltpu.make_async_copy(nq_t.at[idxb.at[1, j]], qr, gs).wait()
        pltpu.make_async_copy(kv_t.at[idxb.at[0, j]], kv, gs).wait()

    def wait_scatter(ct, ss):
        pltpu.make_async_copy(zz_hbm.at[pl.ds(0, CB)], ct, ss).wait()

    def compute(j, qr, kv, ct):
        def group(g, c2):
            pvec = iota + g * LANES
            kvec = idxb[3, j, pl.ds(pl.multiple_of(g * LANES, LANES), LANES)]

            def head(h, c3):
                base = h * D
                accv = jnp.zeros((LANES,), jnp.float32)
                for cc in range(D):
                    col = ((iota + cc) & (D - 1)) + base
                    vq = plsc.load_gather(qr, [pvec, col])
                    vk = plsc.load_gather(kv, [pvec, col])
                    vp = plsc.load_gather(nposv, [kvec, col])
                    accv = accv + vq * (vk + vp)
                for cc in range(D):
                    col2 = ((iota + cc) & (D - 1)) + base
                    colv = col2 + HC
                    vv = plsc.load_gather(kv, [pvec, colv])
                    plsc.store_scatter(ct, [pvec, col2], accv * vv)
                return c3

            lax.fori_loop(0, HPC, head, 0)
            return c2

        lax.fori_loop(0, NGRP, group, 0)

    bufs = ((qrA, kvA, gsA), (qrB, kvB, gsB))

    def super_body(sb, carry):
        pltpu.sync_copy(idx_hbm.at[s, sb], idxb)

        @pl.when(sb > 0)
        def _():
            wait_scatter(ct, ss)

        issue_gather(0, qrA, kvA, gsA)

        def inner(j2, c):
            for b in range(2):
                qr, kv, gs = bufs[b]
                oqr, okv, ogs = bufs[1 - b]
                j = j2 * 2 + b
                wait_gather(j, qr, kv, gs)
                if b == 0:
                    issue_gather(j + 1, oqr, okv, ogs)
                else:
                    @pl.when(j2 < SB // 2 - 1)
                    def _():
                        issue_gather(j + 1, oqr, okv, ogs)

                if b == 1:
                    wait_scatter(ct, ss)
                else:
                    @pl.when(j2 >= 1)
                    def _():
                        wait_scatter(ct, ss)

                compute(j, qr, kv, ct)
                pltpu.async_copy(ct, acc.at[idxb.at[2, j]], ss, add=True)
            return c

        lax.fori_loop(0, SB // 2, inner, 0)
        return carry

    lax.fori_loop(0, NSB, super_body, 0)
    wait_scatter(ct, ss)
    plsc.subcore_barrier()
    pltpu.sync_copy(acc.at[pl.ds(s * RPT, RPT)],
                    o_hbm.at[hh, pl.ds(s * RPT, RPT)])


def _attn(nq3, kv3, npos3, idx5, zz):
    f32 = jnp.float32
    fn = pl.kernel(
        _sc_body,
        out_type=jax.ShapeDtypeStruct((NCORE, NP, HC), f32),
        mesh=plsc.VectorSubcoreMesh(core_axis_name="c", subcore_axis_name="s"),
        compiler_params=pltpu.CompilerParams(needs_layout_passes=False),
        scratch_types=[
            pltpu.VMEM((CB, HC), f32),
            pltpu.VMEM((CB, HC), f32),
            pltpu.VMEM((CB, 2 * HC), f32),
            pltpu.VMEM((CB, 2 * HC), f32),
            pltpu.VMEM((CB, HC), f32),
            pltpu.VMEM((KV, HC), f32),
            pltpu.VMEM((4, SB, CB), jnp.int32),
            pltpu.VMEM_SHARED((NP, HC), f32),
            pltpu.SemaphoreType.DMA,
            pltpu.SemaphoreType.DMA,
            pltpu.SemaphoreType.DMA,
        ],
    )
    return fn(nq3, kv3, npos3, idx5, zz)


def _out_body(o0_ref, o1_ref, wo_ref, bo_ref, x_ref, y_ref):
    oc = jnp.concatenate([o0_ref[...].reshape(RBLK, HC),
                          o1_ref[...].reshape(RBLK, HC)], axis=1)
    y_ref[...] = (jnp.dot(oc, wo_ref[...], preferred_element_type=jnp.float32)
                  + bo_ref[...] + x_ref[...])


def _outproj(out2, wo, bo2, x):
    return pl.pallas_call(
        _out_body,
        grid=(NRB,),
        in_specs=[
            pl.BlockSpec((1, RBLK, HC), lambda i: (0, i, 0)),
            pl.BlockSpec((1, RBLK, HC), lambda i: (1, i, 0)),
            pl.BlockSpec((C, C), lambda i: (0, 0)),
            pl.BlockSpec((1, C), lambda i: (0, 0)),
            pl.BlockSpec((RBLK, C), lambda i: (i, 0)),
        ],
        out_specs=pl.BlockSpec((RBLK, C), lambda i: (i, 0)),
        out_shape=jax.ShapeDtypeStruct((N, C), jnp.float32),
    )(out2, out2, wo, bo2, x)


def kernel(x, kq_indices, kernel_idx, Wq, bq, Wk, bk, Wv, bv, Wo, bo, pos_enc):
    f32 = jnp.float32
    npad = PPTP - PPT
    in_t = kq_indices[0].reshape(NSUB, PPT)
    out_t = kq_indices[1].reshape(NSUB, PPT)
    kid_t = kernel_idx.reshape(NSUB, PPT)
    zpad = jnp.zeros((NSUB, npad), jnp.int32)
    in4 = jnp.concatenate([in_t, zpad], axis=1).reshape(NSUB, NSB, SB, CB)
    og4 = jnp.concatenate([out_t, zpad], axis=1).reshape(NSUB, NSB, SB, CB)
    os4 = jnp.concatenate([out_t, jnp.full((NSUB, npad), N, jnp.int32)],
                          axis=1).reshape(NSUB, NSB, SB, CB)
    kidx4 = jnp.concatenate([kid_t, zpad], axis=1).reshape(NSUB, NSB, SB, CB)
    idx5 = jnp.stack([in4, og4, os4, kidx4], axis=2)
    pos = pos_enc.reshape(KV, C)
    m = jnp.kron(jnp.eye(HPC, dtype=f32), jnp.ones((D, D), f32))
    nq2, kv2, npos2 = _proj(x, Wq, bq.reshape(NCORE, 1, HC), Wk,
                            bk.reshape(NCORE, 1, HC), Wv,
                            bv.reshape(NCORE, 1, HC), pos, m)
    zz = jnp.zeros((NP, HC), f32)
    out2 = _attn(nq2, kv2, npos2, idx5, zz)
    return _outproj(out2, Wo, bo.reshape(1, C), x)
